# baseline (speedup 1.0000x reference)
import functools

import jax
import jax.numpy as jnp
from jax import lax
from jax.experimental import pallas as pl
from jax.experimental.pallas import tpu as pltpu
from jax.experimental.pallas import tpu_sc as plsc

N_FIELDS = 26
VOCAB = 100000
DIM = 32
EPS = 1e-5
OUT_D = N_FIELDS * DIM

L = 16
NC = 2
NS = 16
NW = NC * NS

SEG_LO = (0, 33280, 66560)
SEG_LEN = (33280, 33280, VOCAB - 66560)
NSEG = 3
SEG_BUF = max(SEG_LEN)


def _make_sc_gather(B):
    assert DIM == NW
    NVEC = B // L
    UNROLL = 8
    mesh = plsc.VectorSubcoreMesh(core_axis_name="c", subcore_axis_name="s")

    @functools.partial(
        pl.kernel,
        mesh=mesh,
        compiler_params=pltpu.CompilerParams(
            needs_layout_passes=False, use_tc_tiling_on_sc=True),
        out_type=jax.ShapeDtypeStruct((OUT_D, B), jnp.float32),
        scratch_types=[
            pltpu.VMEM((SEG_BUF,), jnp.float32),
            pltpu.VMEM((SEG_BUF,), jnp.float32),
            pltpu.VMEM((B,), jnp.int32),
            pltpu.VMEM((B,), jnp.int32),
            pltpu.VMEM((B,), jnp.float32),
            pltpu.SemaphoreType.DMA,
            pltpu.SemaphoreType.DMA,
            pltpu.SemaphoreType.DMA,
        ],
    )
    def sc_gather(tabT, catT, out, seg_a, seg_b, idx_a, idx_b, res_v,
                  seg_sem, idx_sem, out_sem):
        d = lax.axis_index("s") * NC + lax.axis_index("c")
        iota = lax.iota(jnp.int32, L)
        segs = (seg_a, seg_b)
        idxs = (idx_a, idx_b)

        def seg_copy(f, s, slot):
            return pltpu.make_async_copy(
                tabT.at[f, d, pl.ds(SEG_LO[s], SEG_LEN[s])],
                segs[slot].at[pl.ds(0, SEG_LEN[s])],
                seg_sem)

        def field(f, ixsl, spat, nspat0):
            pltpu.make_async_copy(catT.at[f], idxs[ixsl], idx_sem).wait()

            @pl.when(f < N_FIELDS - 1)
            def _():
                pltpu.async_copy(catT.at[f + 1], idxs[1 - ixsl], idx_sem)

            @pl.when(f > 0)
            def _():
                pltpu.make_async_copy(res_v, out.at[0], out_sem).wait()

            for s in range(NSEG):
                slot = spat[s]
                seg_copy(f, s, slot).wait()
                if s < NSEG - 1:
                    seg_copy(f, s + 1, spat[s + 1]).start()
                else:
                    @pl.when(f < N_FIELDS - 1)
                    def _():
                        seg_copy(f + 1, 0, nspat0).start()

                lo = SEG_LO[s]
                hi = lo + SEG_LEN[s]

                def sweep(i, carry, slot=slot, lo=lo, hi=hi, s=s):
                    for u in range(UNROLL):
                        j = i * UNROLL + u
                        iv = idxs[ixsl][pl.ds(j * L, L)]
                        if s == 0:
                            m = iv < hi
                        elif s == NSEG - 1:
                            m = iv >= lo
                        else:
                            m = (iv >= lo) & (iv < hi)
                        lv = jnp.minimum(
                            jnp.maximum(iv - lo, 0), SEG_LEN[s] - 1)
                        g = plsc.load_gather(segs[slot], [lv], mask=m)
                        plsc.store_scatter(res_v, [iota + j * L], g, mask=m)
                    return carry

                lax.fori_loop(0, NVEC // UNROLL, sweep, 0)

            pltpu.async_copy(res_v, out.at[f * DIM + d], out_sem)

        pltpu.async_copy(catT.at[0], idx_a, idx_sem)
        seg_copy(0, 0, 0).start()

        def gbody(g, carry):
            field(2 * g, 0, (0, 1, 0), 1)
            field(2 * g + 1, 1, (1, 0, 1), 0)
            return carry

        lax.fori_loop(0, N_FIELDS // 2, gbody, 0)
        pltpu.make_async_copy(res_v, out.at[0], out_sem).wait()

    return sc_gather


def _tc_layernorm(gath, gamma, beta):
    D, B = gath.shape
    BL = 512

    def ln_body(x_ref, g_ref, b_ref, o_ref):
        x = x_ref[...]
        mean = jnp.mean(x, axis=0, keepdims=True)
        xc = x - mean
        var = jnp.mean(xc * xc, axis=0, keepdims=True)
        r = lax.rsqrt(var + EPS)
        o_ref[...] = xc * r * g_ref[...] + b_ref[...]

    return pl.pallas_call(
        ln_body,
        grid=(B // BL,),
        in_specs=[
            pl.BlockSpec((D, BL), lambda i: (0, i)),
            pl.BlockSpec((D, 1), lambda i: (0, 0)),
            pl.BlockSpec((D, 1), lambda i: (0, 0)),
        ],
        out_specs=pl.BlockSpec((D, BL), lambda i: (0, i)),
        out_shape=jax.ShapeDtypeStruct((D, B), jnp.float32),
    )(gath, gamma.reshape(D, 1), beta.reshape(D, 1))


def kernel(cat, tables, gamma, beta):
    B = cat.shape[0]
    catT = cat.T
    tabT = tables.transpose(0, 2, 1)
    gath = _make_sc_gather(B)(tabT, catT)
    outT = _tc_layernorm(gath, gamma, beta)
    return outT.T


# speedup vs baseline: 2.1120x; 2.1120x over previous
import functools

import jax
import jax.numpy as jnp
from jax import lax
from jax.experimental import pallas as pl
from jax.experimental.pallas import tpu as pltpu
from jax.experimental.pallas import tpu_sc as plsc

N_FIELDS = 26
VOCAB = 100000
DIM = 32
EPS = 1e-5
OUT_D = N_FIELDS * DIM

L = 16
NC = 2
NS = 16
NW = NC * NS

SEG0 = 50048
SEG1 = VOCAB - SEG0


def _make_sc_gather(B):
    assert DIM == NW
    NVEC = B // L
    UNROLL = 4
    mesh = plsc.VectorSubcoreMesh(core_axis_name="c", subcore_axis_name="s")

    @functools.partial(
        pl.kernel,
        mesh=mesh,
        compiler_params=pltpu.CompilerParams(
            needs_layout_passes=False, use_tc_tiling_on_sc=True),
        out_type=jax.ShapeDtypeStruct((OUT_D, B), jnp.float32),
        scratch_types=[
            pltpu.VMEM((SEG0,), jnp.float32),
            pltpu.VMEM((SEG1,), jnp.float32),
            pltpu.VMEM((B,), jnp.float32),
            pltpu.SemaphoreType.DMA,
            pltpu.SemaphoreType.DMA,
            pltpu.SemaphoreType.DMA,
        ],
    )
    def sc_gather(tabT, catTf, out, seg_a, seg_b, res_v,
                  seg_sem, idx_sem, out_sem):
        d = lax.axis_index("s") * NC + lax.axis_index("c")

        def seg0_copy(f):
            return pltpu.make_async_copy(
                tabT.at[f, d, pl.ds(0, SEG0)], seg_a, seg_sem)

        def seg1_copy(f):
            return pltpu.make_async_copy(
                tabT.at[f, d, pl.ds(SEG0, SEG1)], seg_b, seg_sem)

        def sweep(s):
            seg = (seg_a, seg_b)[s]

            def body(i, carry):
                for u in range(UNROLL):
                    j = i * UNROLL + u
                    sl = pl.ds(j * L, L)
                    v = res_v[sl]
                    iv = plsc.bitcast(v, jnp.int32)
                    if s == 0:
                        m = iv < SEG0
                        lv = jnp.minimum(iv, SEG0 - 1)
                    else:
                        m = iv >= SEG0
                        lv = jnp.minimum(
                            jnp.maximum(iv - SEG0, 0), SEG1 - 1)
                    g = plsc.load_gather(seg, [lv])
                    res_v[sl] = jnp.where(m, g, v)
                return carry

            lax.fori_loop(0, NVEC // UNROLL, body, 0)

        seg0_copy(0).start()

        def fbody(f, carry):
            @pl.when(f > 0)
            def _():
                pltpu.make_async_copy(res_v, out.at[0], out_sem).wait()

            pltpu.async_copy(catTf.at[f], res_v, idx_sem)
            seg0_copy(f).wait()
            seg1_copy(f).start()
            pltpu.make_async_copy(catTf.at[f], res_v, idx_sem).wait()
            sweep(0)

            @pl.when(f < N_FIELDS - 1)
            def _():
                seg0_copy(f + 1).start()

            seg1_copy(f).wait()
            sweep(1)
            pltpu.async_copy(res_v, out.at[f * DIM + d], out_sem)
            return carry

        lax.fori_loop(0, N_FIELDS, fbody, 0)
        pltpu.make_async_copy(res_v, out.at[0], out_sem).wait()

    return sc_gather


def _tc_layernorm(gath, gamma, beta):
    D, B = gath.shape
    BL = 512

    def ln_body(x_ref, g_ref, b_ref, o_ref):
        x = x_ref[...]
        mean = jnp.mean(x, axis=0, keepdims=True)
        xc = x - mean
        var = jnp.mean(xc * xc, axis=0, keepdims=True)
        r = lax.rsqrt(var + EPS)
        o_ref[...] = xc * r * g_ref[...] + b_ref[...]

    return pl.pallas_call(
        ln_body,
        grid=(B // BL,),
        in_specs=[
            pl.BlockSpec((D, BL), lambda i: (0, i)),
            pl.BlockSpec((D, 1), lambda i: (0, 0)),
            pl.BlockSpec((D, 1), lambda i: (0, 0)),
        ],
        out_specs=pl.BlockSpec((D, BL), lambda i: (0, i)),
        out_shape=jax.ShapeDtypeStruct((D, B), jnp.float32),
    )(gath, gamma.reshape(D, 1), beta.reshape(D, 1))


def kernel(cat, tables, gamma, beta):
    B = cat.shape[0]
    catTf = lax.bitcast_convert_type(cat.T, jnp.float32)
    tabT = tables.transpose(0, 2, 1)
    gath = _make_sc_gather(B)(tabT, catTf)
    outT = _tc_layernorm(gath, gamma, beta)
    return outT.T
